# baseline (speedup 1.0000x reference)
import jax
import jax.numpy as jnp
from jax.experimental import pallas as pl
from jax.experimental.pallas import tpu as pltpu

_EPS = 1e-6
_D = 256
_CHUNK = 1024


def _mcl_kernel(f_hbm, c_hbm, l_ref, out_ref, fbuf, cbuf, csem, fsem):
    batch = f_hbm.shape[0]
    nchunk = batch // _CHUNK

    c_copy = pltpu.make_async_copy(c_hbm, cbuf, csem)
    c_copy.start()
    copies = [
        pltpu.make_async_copy(
            f_hbm.at[pl.ds(k * _CHUNK, _CHUNK), :], fbuf.at[k % 2], fsem.at[k % 2]
        )
        for k in range(nchunk)
    ]
    copies[0].start()
    c_copy.wait()

    c = cbuf[...]
    cs = -2.0 * c.astype(jnp.bfloat16)
    cn = (jnp.sum(c * c, axis=1) - (2.0 * _EPS) * jnp.sum(c, axis=1))[
        None, :
    ]

    acc_s = jnp.float32(0.0)
    acc_n = jnp.float32(0.0)
    for k in range(nchunk):
        if k + 1 < nchunk:
            copies[k + 1].start()
        copies[k].wait()
        f = fbuf[k % 2]
        dot = jax.lax.dot_general(
            f.astype(jnp.bfloat16), cs, (((1,), (1,)), ((), ())),
            preferred_element_type=jnp.float32,
        )
        t = dot + cn
        m = jnp.min(t, axis=1, keepdims=True)
        rn = jnp.sum(f * f, axis=1, keepdims=True) + (2.0 * _EPS) * jnp.sum(
            f, axis=1, keepdims=True
        )
        min_d = jnp.sqrt(jnp.maximum(m + rn + (_D * _EPS * _EPS), 0.0))
        mask = (l_ref[pl.ds(k * _CHUNK, _CHUNK), :] == 0).astype(jnp.float32)
        acc_s += jnp.sum(mask * min_d)
        acc_n += jnp.sum(mask)

    out_ref[0, 0] = acc_s / (acc_n + 1e-5)


def kernel(features, labels, centers):
    batch, d = features.shape
    ncenters = centers.shape[0]
    labels2 = labels.reshape(batch, 1)
    out = pl.pallas_call(
        _mcl_kernel,
        grid=(1,),
        in_specs=[
            pl.BlockSpec(memory_space=pltpu.MemorySpace.HBM),
            pl.BlockSpec(memory_space=pltpu.MemorySpace.HBM),
            pl.BlockSpec((batch, 1), lambda i: (0, 0)),
        ],
        out_specs=pl.BlockSpec(
            (1, 1), lambda i: (0, 0), memory_space=pltpu.SMEM
        ),
        out_shape=jax.ShapeDtypeStruct((1, 1), jnp.float32),
        scratch_shapes=[
            pltpu.VMEM((2, _CHUNK, d), jnp.float32),
            pltpu.VMEM((ncenters, d), jnp.float32),
            pltpu.SemaphoreType.DMA,
            pltpu.SemaphoreType.DMA((2,)),
        ],
    )(features, centers, labels2)
    return out[0, 0]


# speedup vs baseline: 1.7963x; 1.7963x over previous
import jax
import jax.numpy as jnp
from jax.experimental import pallas as pl
from jax.experimental.pallas import tpu as pltpu

_EPS = 1e-6
_D = 256


def _mcl_kernel(f_ref, c_ref, l_ref, out_ref, acc_s, acc_n):
    c = c_ref[...]
    cs = -2.0 * c.astype(jnp.bfloat16)
    cn = (jnp.sum(c * c, axis=1) - (2.0 * _EPS) * jnp.sum(c, axis=1))[
        :, None
    ]

    f = f_ref[...]
    dot = jax.lax.dot_general(
        cs, f.astype(jnp.bfloat16), (((1,), (1,)), ((), ())),
        preferred_element_type=jnp.float32,
    )
    t = dot + cn
    m = jnp.min(t, axis=0, keepdims=True)
    ones_d = jnp.ones((1, f.shape[1]), jnp.float32)
    rn = jax.lax.dot_general(
        ones_d, f * (f + 2.0 * _EPS), (((1,), (1,)), ((), ())),
        preferred_element_type=jnp.float32,
    )
    min_d = jnp.sqrt(jnp.maximum(m + rn + (_D * _EPS * _EPS), 0.0))
    mask = (l_ref[...] == 0).astype(jnp.float32)
    acc_s[0, 0] = jnp.sum(mask * min_d)
    acc_n[0, 0] = jnp.sum(mask)
    out_ref[0, 0] = acc_s[0, 0] / (acc_n[0, 0] + 1e-5)


def kernel(features, labels, centers):
    batch, d = features.shape
    ncenters = centers.shape[0]
    labels2 = labels.reshape(1, batch)
    out = pl.pallas_call(
        _mcl_kernel,
        grid=(1,),
        in_specs=[
            pl.BlockSpec((batch, d), lambda i: (0, 0)),
            pl.BlockSpec((ncenters, d), lambda i: (0, 0)),
            pl.BlockSpec((1, batch), lambda i: (0, 0)),
        ],
        out_specs=pl.BlockSpec(
            (1, 1), lambda i: (0, 0), memory_space=pltpu.SMEM
        ),
        out_shape=jax.ShapeDtypeStruct((1, 1), jnp.float32),
        scratch_shapes=[
            pltpu.SMEM((1, 1), jnp.float32),
            pltpu.SMEM((1, 1), jnp.float32),
        ],
    )(features, centers, labels2)
    return out[0, 0]
